# Initial kernel scaffold; baseline (speedup 1.0000x reference)
import jax
import jax.numpy as jnp
from jax.experimental import pallas as pl


def kernel(x, edge_index, W1, b1, W2, b2):
    raise NotImplementedError("write your pallas kernel here")


# speedup vs baseline: 15.8006x; 15.8006x over previous
import functools

import jax
import jax.numpy as jnp
from jax import lax
from jax.experimental import pallas as pl
from jax.experimental.pallas import tpu as pltpu
from jax.experimental.pallas import tpu_sc as plsc

N = 10000
D = 128
NC, NS = 2, 16
NW = NC * NS
NPAD = 10240
RPT = NPAD // NS
K = 128
E = 320000
EPW_CHUNKS = -(-E // (NW * K))
EPW = EPW_CHUNKS * K
E_PAD = EPW * NW

_mesh = plsc.VectorSubcoreMesh(core_axis_name="c", subcore_axis_name="s")


@functools.partial(
    pl.kernel,
    out_type=jax.ShapeDtypeStruct((NC, 2, NPAD), jnp.float32),
    mesh=_mesh,
    scratch_types=[
        pltpu.VMEM((K,), jnp.int32),
        pltpu.VMEM((K,), jnp.float32),
        pltpu.VMEM((RPT,), jnp.float32),
        pltpu.VMEM_SHARED((NPAD,), jnp.float32),
        pltpu.VMEM_SHARED((NPAD,), jnp.float32),
    ],
)
def _sc_degrees(src_hbm, dst_hbm, ones_hbm, zer_hbm, out_hbm,
                idx_v, ones_v, buf_v, dsrc_sh, ddst_sh):
    c = lax.axis_index("c")
    s = lax.axis_index("s")
    wid = s * NC + c
    pltpu.sync_copy(ones_hbm, ones_v)
    pltpu.sync_copy(zer_hbm, buf_v)
    r0 = s * RPT
    pltpu.sync_copy(buf_v, dsrc_sh.at[pl.ds(r0, RPT)])
    pltpu.sync_copy(buf_v, ddst_sh.at[pl.ds(r0, RPT)])
    plsc.subcore_barrier()
    e0 = wid * EPW

    def body(i, carry):
        base = e0 + i * K
        pltpu.sync_copy(src_hbm.at[pl.ds(base, K)], idx_v)
        pltpu.sync_copy(ones_v, dsrc_sh.at[idx_v], add=True)
        pltpu.sync_copy(dst_hbm.at[pl.ds(base, K)], idx_v)
        pltpu.sync_copy(ones_v, ddst_sh.at[idx_v], add=True)
        return carry

    lax.fori_loop(0, EPW_CHUNKS, body, 0)
    plsc.subcore_barrier()
    pltpu.sync_copy(dsrc_sh.at[pl.ds(r0, RPT)], buf_v)
    pltpu.sync_copy(buf_v, out_hbm.at[c, 0, pl.ds(r0, RPT)])
    pltpu.sync_copy(ddst_sh.at[pl.ds(r0, RPT)], buf_v)
    pltpu.sync_copy(buf_v, out_hbm.at[c, 1, pl.ds(r0, RPT)])


@functools.partial(
    pl.kernel,
    out_type=jax.ShapeDtypeStruct((NC, NPAD, D), jnp.float32),
    mesh=_mesh,
    scratch_types=[
        pltpu.VMEM((K,), jnp.int32),
        pltpu.VMEM((K,), jnp.int32),
        pltpu.VMEM((K, D), jnp.float32),
        pltpu.VMEM_SHARED((NPAD, D), jnp.float32),
        pltpu.SemaphoreType.DMA,
    ],
)
def _sc_edge_scatter(hs_hbm, src_hbm, dst_hbm, zrows_hbm, out_hbm,
                     sidx, didx, rows, acc, sem):
    c = lax.axis_index("c")
    s = lax.axis_index("s")
    wid = s * NC + c
    r0 = s * RPT
    pltpu.sync_copy(zrows_hbm, rows)
    for z in range(RPT // K):
        pltpu.sync_copy(rows, acc.at[pl.ds(r0 + z * K, K)])
    plsc.subcore_barrier()
    e0 = wid * EPW

    def body(i, carry):
        base = e0 + i * K
        pltpu.sync_copy(src_hbm.at[pl.ds(base, K)], sidx)
        pltpu.async_copy(hs_hbm.at[sidx], rows, sem).wait()
        pltpu.sync_copy(dst_hbm.at[pl.ds(base, K)], didx)
        pltpu.sync_copy(rows, acc.at[didx], add=True)
        return carry

    lax.fori_loop(0, EPW_CHUNKS, body, 0)
    plsc.subcore_barrier()
    for z in range(RPT // K):
        pltpu.sync_copy(acc.at[pl.ds(r0 + z * K, K)], rows)
        pltpu.sync_copy(rows, out_hbm.at[c, pl.ds(r0 + z * K, K)])


RB = 1024


def _rsqrt_a(p_ref):
    return lax.rsqrt(p_ref[:, 0:1] + p_ref[:, 2:3] + 1.0)


def _rsqrt_c(p_ref):
    return lax.rsqrt(p_ref[:, 1:2] + p_ref[:, 3:4] + 1.0)


def _tc_pre_body(x_ref, w_ref, p_ref, o_ref):
    h = jnp.dot(x_ref[...], w_ref[...], preferred_element_type=jnp.float32)
    o_ref[...] = h * _rsqrt_a(p_ref)


_tc_pre = pl.pallas_call(
    _tc_pre_body,
    grid=(NPAD // RB,),
    in_specs=[
        pl.BlockSpec((RB, D), lambda i: (i, 0)),
        pl.BlockSpec((D, D), lambda i: (0, 0)),
        pl.BlockSpec((RB, 4), lambda i: (i, 0)),
    ],
    out_specs=pl.BlockSpec((RB, D), lambda i: (i, 0)),
    out_shape=jax.ShapeDtypeStruct((NPAD, D), jnp.float32),
)


def _tc_mid_body(s_ref, hs_ref, p_ref, b_ref, w_ref, o_ref):
    tot = s_ref[0] + s_ref[1] + hs_ref[...]
    h1 = jnp.maximum(_rsqrt_c(p_ref) * tot + b_ref[...], 0.0)
    h2 = jnp.dot(h1, w_ref[...], preferred_element_type=jnp.float32)
    o_ref[...] = h2 * _rsqrt_a(p_ref)


_tc_mid = pl.pallas_call(
    _tc_mid_body,
    grid=(NPAD // RB,),
    in_specs=[
        pl.BlockSpec((NC, RB, D), lambda i: (0, i, 0)),
        pl.BlockSpec((RB, D), lambda i: (i, 0)),
        pl.BlockSpec((RB, 4), lambda i: (i, 0)),
        pl.BlockSpec((1, D), lambda i: (0, 0)),
        pl.BlockSpec((D, D), lambda i: (0, 0)),
    ],
    out_specs=pl.BlockSpec((RB, D), lambda i: (i, 0)),
    out_shape=jax.ShapeDtypeStruct((NPAD, D), jnp.float32),
)


def _tc_fin_body(s_ref, hs_ref, p_ref, b_ref, o_ref):
    tot = s_ref[0] + s_ref[1] + hs_ref[...]
    o_ref[...] = _rsqrt_c(p_ref) * tot + b_ref[...]


_tc_fin = pl.pallas_call(
    _tc_fin_body,
    grid=(NPAD // RB,),
    in_specs=[
        pl.BlockSpec((NC, RB, D), lambda i: (0, i, 0)),
        pl.BlockSpec((RB, D), lambda i: (i, 0)),
        pl.BlockSpec((RB, 4), lambda i: (i, 0)),
        pl.BlockSpec((1, D), lambda i: (0, 0)),
    ],
    out_specs=pl.BlockSpec((RB, D), lambda i: (i, 0)),
    out_shape=jax.ShapeDtypeStruct((NPAD, D), jnp.float32),
)


def kernel(x, edge_index, W1, b1, W2, b2):
    ei = edge_index.astype(jnp.int32)
    pad_idx = N + (jnp.arange(E_PAD - E, dtype=jnp.int32) % (NPAD - N))
    src = jnp.concatenate([ei[0], pad_idx])
    dst = jnp.concatenate([ei[1], pad_idx])
    x_pad = jnp.concatenate([x, jnp.zeros((NPAD - N, D), jnp.float32)], axis=0)
    ones_k = jnp.ones((K,), jnp.float32)
    zer_r = jnp.zeros((RPT,), jnp.float32)
    zrows = jnp.zeros((K, D), jnp.float32)

    degs = _sc_degrees(src, dst, ones_k, zer_r)
    p = jnp.transpose(degs.reshape(2 * NC, NPAD))

    hs1 = _tc_pre(x_pad, W1, p)
    s1 = _sc_edge_scatter(hs1, src, dst, zrows)
    hs2 = _tc_mid(s1, hs1, p, b1.reshape(1, D), W2)
    s2 = _sc_edge_scatter(hs2, src, dst, zrows)
    out = _tc_fin(s2, hs2, p, b2.reshape(1, D))
    return out[:N][None]
